# baseline (speedup 1.0000x reference)
import functools

import jax
import jax.numpy as jnp
from jax import lax
from jax.experimental import pallas as pl
from jax.experimental.pallas import tpu as pltpu
from jax.experimental.pallas import tpu_sc as plsc

_B, _N = 8, 262144
_LANES = 128
_NH = _N // _LANES
_CH = 64

_NH_SC = 2048
_TECS = 32
_SC_NH_PER = _NH_SC // _TECS if _NH_SC else 0
_SC_IT = _SC_NH_PER // 8 if _NH_SC else 0

_P6 = (-0.017414077524383682, 0.08269123711190571, -0.1903543367337465,
       0.3157473167583923, -0.4973732161580652, 0.9998476974962458,
       1.472065011022741e-06)


def _tc_body(lab_ref, lg_ref, w_ref, out_ref, acc_ref):
    i = pl.program_id(0)

    @pl.when(i == 0)
    def _():
        acc_ref[...] = jnp.zeros_like(acc_ref)

    for s in range(_CH // 8):
        l0 = lg_ref[:, pl.Slice(16 * s, 8, 2), :]
        l1 = lg_ref[:, pl.Slice(16 * s + 1, 8, 2), :]
        lab = lab_ref[:, pl.ds(1024 * s, 1024)].reshape(_B, 8, _LANES)
        w = w_ref[:, pl.ds(1024 * s, 1024)].reshape(_B, 8, _LANES)
        d = l1 - l0
        sp = jnp.log1p(jnp.exp(-jnp.abs(d)))
        ce = jnp.maximum(d, 0.0) - lab.astype(jnp.float32) * d + sp
        wm = jnp.where(lab != -1, w, 0.0)
        acc_ref[...] += jnp.sum(ce * wm, axis=1)

    @pl.when(i == pl.num_programs(0) - 1)
    def _():
        out_ref[0, 0] = jnp.sum(acc_ref[...])


def _tc_part(rpn_labels, lg_v, rpn_label_weights):
    off = _NH_SC // _CH
    grid = ((_NH - _NH_SC) // _CH,)
    out = pl.pallas_call(
        _tc_body,
        grid=grid,
        in_specs=[
            pl.BlockSpec((_B, _CH * _LANES), lambda i: (0, i + off)),
            pl.BlockSpec((_B, 2 * _CH, _LANES), lambda i: (0, i + off, 0)),
            pl.BlockSpec((_B, _CH * _LANES), lambda i: (0, i + off)),
        ],
        out_specs=pl.BlockSpec(memory_space=pltpu.SMEM),
        out_shape=jax.ShapeDtypeStruct((1, 1), jnp.float32),
        scratch_shapes=[pltpu.VMEM((_B, _LANES), jnp.float32)],
    )(rpn_labels, lg_v, rpn_label_weights)
    return out[0, 0]


def _sc_fire(lab_hbm, lg_hbm, w_hbm, labd, lgd, wd, sem, nh0):
    hs = [pltpu.async_copy(lab_hbm.at[pl.ds(nh0 * 8, 64)], labd, sem),
          pltpu.async_copy(w_hbm.at[pl.ds(nh0 * 8, 64)], wd, sem)]
    for b in range(8):
        hs.append(pltpu.async_copy(lg_hbm.at[b, pl.ds(nh0 * 2, 16)],
                                   lgd.at[b], sem))
    return hs


def _sc_body(lab_hbm, lg_hbm, w_hbm, out_hbm,
             labb, lgb, wb, acc, sem0, sem1, osem):
    wid = lax.axis_index("c") * 16 + lax.axis_index("s")
    nh_base = wid * _SC_NH_PER
    sems = (sem0, sem1)

    def compute(labB, lgB, wB, accs):
        def trip(i, accs):
            nh = lax.shift_right_logical(i, 3)
            b = lax.bitwise_and(i, 7)
            row = i
            outs = []
            for k in range(8):
                lo = lgB[b, 2 * nh, pl.ds(k * 16, 16)]
                l1 = lgB[b, 2 * nh + 1, pl.ds(k * 16, 16)]
                lv = labB[row, pl.ds(k * 16, 16)]
                wv = wB[row, pl.ds(k * 16, 16)]
                d = l1 - lo
                nd = -d
                y = jnp.exp(jnp.minimum(d, nd))
                sp = jnp.float32(_P6[0])
                for c in _P6[1:]:
                    sp = sp * y + jnp.float32(c)
                s = jnp.where(lv == 1, nd, d)
                ce = jnp.maximum(s, 0.0) + sp
                wm = jnp.where(lv != -1, wv, 0.0)
                outs.append(accs[k] + ce * wm)
            return tuple(outs)
        return lax.fori_loop(0, 64, trip, accs)

    accs = tuple(jnp.zeros((16,), jnp.float32) for _ in range(8))
    handles = _sc_fire(lab_hbm, lg_hbm, w_hbm,
                       labb.at[0], lgb.at[0], wb.at[0], sems[0], nh_base)
    for it in range(_SC_IT):
        cur = it % 2
        nxt = None
        if it + 1 < _SC_IT:
            nb = (it + 1) % 2
            nxt = _sc_fire(lab_hbm, lg_hbm, w_hbm,
                           labb.at[nb], lgb.at[nb], wb.at[nb], sems[nb],
                           nh_base + (it + 1) * 8)
        for h in handles:
            h.wait()
        accs = compute(labb.at[cur], lgb.at[cur], wb.at[cur], accs)
        handles = nxt

    t01 = accs[0] + accs[1]
    t23 = accs[2] + accs[3]
    t45 = accs[4] + accs[5]
    t67 = accs[6] + accs[7]
    acc[...] = (t01 + t23) + (t45 + t67)
    pltpu.async_copy(acc, out_hbm.at[pl.ds(wid * 16, 16)], osem).wait()


def _sc_part(lab_sc, lg_sc, w_sc):
    kern = pl.kernel(
        _sc_body,
        out_type=jax.ShapeDtypeStruct((_TECS * 16,), jnp.float32),
        mesh=plsc.VectorSubcoreMesh(core_axis_name="c", subcore_axis_name="s"),
        scratch_types=[
            pltpu.VMEM((2, 64, 128), jnp.int32),
            pltpu.VMEM((2, 8, 16, 128), jnp.float32),
            pltpu.VMEM((2, 64, 128), jnp.float32),
            pltpu.VMEM((16,), jnp.float32),
            pltpu.SemaphoreType.DMA,
            pltpu.SemaphoreType.DMA,
            pltpu.SemaphoreType.DMA,
        ],
    )
    return jnp.sum(kern(lab_sc, lg_sc, w_sc))


def kernel(rpn_labels, rpn_class_logits, rpn_label_weights):
    lg_v = rpn_class_logits.reshape(_B, _NH, _LANES, 2)
    lg_v = lg_v.transpose(0, 1, 3, 2).reshape(_B, 2 * _NH, _LANES)

    parts = []
    if _NH_SC > 0:
        lab_sc = rpn_labels.reshape(_B, _NH, _LANES)
        lab_sc = lab_sc.transpose(1, 0, 2).reshape(_NH * _B, _LANES)
        w_sc = rpn_label_weights.reshape(_B, _NH, _LANES)
        w_sc = w_sc.transpose(1, 0, 2).reshape(_NH * _B, _LANES)
        parts.append(_sc_part(lab_sc, lg_v, w_sc))
    if _NH_SC < _NH:
        parts.append(_tc_part(rpn_labels, lg_v, rpn_label_weights))
    return functools.reduce(jnp.add, parts)


# speedup vs baseline: 1.2161x; 1.2161x over previous
import functools

import jax
import jax.numpy as jnp
from jax import lax
from jax.experimental import pallas as pl
from jax.experimental.pallas import tpu as pltpu
from jax.experimental.pallas import tpu_sc as plsc

_B, _N = 8, 262144
_LANES = 128
_NH = _N // _LANES
_CH = 64

_NH_SC = 512
_TECS = 32
_SC_NH_PER = _NH_SC // _TECS if _NH_SC else 0
_SC_IT = _SC_NH_PER // 8 if _NH_SC else 0

_P6 = (-0.017414077524383682, 0.08269123711190571, -0.1903543367337465,
       0.3157473167583923, -0.4973732161580652, 0.9998476974962458,
       1.472065011022741e-06)


def _tc_body(lab_ref, lg_ref, w_ref, out_ref, acc_ref):
    i = pl.program_id(0)

    @pl.when(i == 0)
    def _():
        acc_ref[...] = jnp.zeros_like(acc_ref)

    for s in range(_CH // 8):
        l0 = lg_ref[:, pl.Slice(16 * s, 8, 2), :]
        l1 = lg_ref[:, pl.Slice(16 * s + 1, 8, 2), :]
        lab = lab_ref[:, pl.ds(1024 * s, 1024)].reshape(_B, 8, _LANES)
        w = w_ref[:, pl.ds(1024 * s, 1024)].reshape(_B, 8, _LANES)
        d = l1 - l0
        sp = jnp.log1p(jnp.exp(-jnp.abs(d)))
        ce = jnp.maximum(d, 0.0) - lab.astype(jnp.float32) * d + sp
        wm = jnp.where(lab != -1, w, 0.0)
        acc_ref[...] += jnp.sum(ce * wm, axis=1)

    @pl.when(i == pl.num_programs(0) - 1)
    def _():
        out_ref[0, 0] = jnp.sum(acc_ref[...])


def _tc_part(rpn_labels, lg_v, rpn_label_weights):
    off = _NH_SC // _CH
    grid = ((_NH - _NH_SC) // _CH,)
    out = pl.pallas_call(
        _tc_body,
        grid=grid,
        in_specs=[
            pl.BlockSpec((_B, _CH * _LANES), lambda i: (0, i + off)),
            pl.BlockSpec((_B, 2 * _CH, _LANES), lambda i: (0, i + off, 0)),
            pl.BlockSpec((_B, _CH * _LANES), lambda i: (0, i + off)),
        ],
        out_specs=pl.BlockSpec(memory_space=pltpu.SMEM),
        out_shape=jax.ShapeDtypeStruct((1, 1), jnp.float32),
        scratch_shapes=[pltpu.VMEM((_B, _LANES), jnp.float32)],
    )(rpn_labels, lg_v, rpn_label_weights)
    return out[0, 0]


def _sc_fire(lab_hbm, lg_hbm, w_hbm, labd, lgd, wd, sem, nh0):
    hs = [pltpu.async_copy(lab_hbm.at[pl.ds(nh0 * 8, 64)], labd, sem),
          pltpu.async_copy(w_hbm.at[pl.ds(nh0 * 8, 64)], wd, sem)]
    for b in range(8):
        hs.append(pltpu.async_copy(lg_hbm.at[b, pl.ds(nh0 * 2, 16)],
                                   lgd.at[b], sem))
    return hs


def _sc_body(lab_hbm, lg_hbm, w_hbm, out_hbm,
             labb, lgb, wb, acc, sem0, sem1, osem):
    wid = lax.axis_index("c") * 16 + lax.axis_index("s")
    nh_base = wid * _SC_NH_PER
    sems = (sem0, sem1)

    def compute(labB, lgB, wB, accs):
        def trip(i, accs):
            nh = lax.shift_right_logical(i, 3)
            b = lax.bitwise_and(i, 7)
            row = i
            outs = []
            for k in range(8):
                lo = lgB[b, 2 * nh, pl.ds(k * 16, 16)]
                l1 = lgB[b, 2 * nh + 1, pl.ds(k * 16, 16)]
                lv = labB[row, pl.ds(k * 16, 16)]
                wv = wB[row, pl.ds(k * 16, 16)]
                d = l1 - lo
                nd = -d
                y = jnp.exp(jnp.minimum(d, nd))
                sp = jnp.float32(_P6[0])
                for c in _P6[1:]:
                    sp = sp * y + jnp.float32(c)
                s = jnp.where(lv == 1, nd, d)
                ce = jnp.maximum(s, 0.0) + sp
                wm = jnp.where(lv != -1, wv, 0.0)
                outs.append(accs[k] + ce * wm)
            return tuple(outs)
        return lax.fori_loop(0, 64, trip, accs)

    accs = tuple(jnp.zeros((16,), jnp.float32) for _ in range(8))
    handles = _sc_fire(lab_hbm, lg_hbm, w_hbm,
                       labb.at[0], lgb.at[0], wb.at[0], sems[0], nh_base)
    for it in range(_SC_IT):
        cur = it % 2
        nxt = None
        if it + 1 < _SC_IT:
            nb = (it + 1) % 2
            nxt = _sc_fire(lab_hbm, lg_hbm, w_hbm,
                           labb.at[nb], lgb.at[nb], wb.at[nb], sems[nb],
                           nh_base + (it + 1) * 8)
        for h in handles:
            h.wait()
        accs = compute(labb.at[cur], lgb.at[cur], wb.at[cur], accs)
        handles = nxt

    t01 = accs[0] + accs[1]
    t23 = accs[2] + accs[3]
    t45 = accs[4] + accs[5]
    t67 = accs[6] + accs[7]
    acc[...] = (t01 + t23) + (t45 + t67)
    pltpu.async_copy(acc, out_hbm.at[pl.ds(wid * 16, 16)], osem).wait()


def _sc_part(lab_sc, lg_sc, w_sc):
    kern = pl.kernel(
        _sc_body,
        out_type=jax.ShapeDtypeStruct((_TECS * 16,), jnp.float32),
        mesh=plsc.VectorSubcoreMesh(core_axis_name="c", subcore_axis_name="s"),
        scratch_types=[
            pltpu.VMEM((2, 64, 128), jnp.int32),
            pltpu.VMEM((2, 8, 16, 128), jnp.float32),
            pltpu.VMEM((2, 64, 128), jnp.float32),
            pltpu.VMEM((16,), jnp.float32),
            pltpu.SemaphoreType.DMA,
            pltpu.SemaphoreType.DMA,
            pltpu.SemaphoreType.DMA,
        ],
    )
    return jnp.sum(kern(lab_sc, lg_sc, w_sc))


def kernel(rpn_labels, rpn_class_logits, rpn_label_weights):
    lg_v = rpn_class_logits.reshape(_B, _NH, _LANES, 2)
    lg_v = lg_v.transpose(0, 1, 3, 2).reshape(_B, 2 * _NH, _LANES)

    parts = []
    if _NH_SC > 0:
        lab_sc = rpn_labels.reshape(_B, _NH, _LANES)
        lab_sc = lab_sc.transpose(1, 0, 2).reshape(_NH * _B, _LANES)
        w_sc = rpn_label_weights.reshape(_B, _NH, _LANES)
        w_sc = w_sc.transpose(1, 0, 2).reshape(_NH * _B, _LANES)
        parts.append(_sc_part(lab_sc, lg_v, w_sc))
    if _NH_SC < _NH:
        parts.append(_tc_part(rpn_labels, lg_v, rpn_label_weights))
    return functools.reduce(jnp.add, parts)
